# baseline (speedup 1.0000x reference)
import functools

import jax
import jax.numpy as jnp
import numpy as np
from jax.experimental import pallas as pl
from jax.experimental.pallas import tpu as pltpu

N_LIN = 8
T_VALS = 8
BLK = 2048


def _celu(h):
    return jnp.where(h > 0, h, jnp.exp(jnp.minimum(h, 0.0)) - 1.0)


def _select(arr, oh8, w):
    out = oh8[:, 0:1] * arr[:, 0:w]
    for i in range(1, N_LIN):
        out = out + oh8[:, i:i + 1] * arr[:, w * i:w * (i + 1)]
    return out


def _body(key_ref, x_ref, v_ref,
          a1g_ref, b1g_ref, a2g_ref, b2g_ref, a3g_ref, b3g_ref,
          a1l_ref, b1l_ref, w2bd_ref, b2bd_ref, a3l_ref, b3l8_ref,
          recon_ref, orth_ref, sim_ref,
          acc_vg, acc_cnt, acc_orth, acc_recon):
    i = pl.program_id(0)
    nb = pl.num_programs(0)
    f32 = jnp.float32

    @pl.when(i == 0)
    def _init():
        acc_vg[...] = jnp.zeros_like(acc_vg)
        acc_cnt[...] = jnp.zeros_like(acc_cnt)
        acc_orth[...] = jnp.zeros_like(acc_orth)
        acc_recon[...] = jnp.zeros_like(acc_recon)

    x = x_ref[...]
    v = v_ref[...]
    key = key_ref[...]
    idx = jnp.bitwise_and(key, N_LIN - 1)

    h = _celu(jnp.dot(x, a1g_ref[...], preferred_element_type=f32) + b1g_ref[...])
    h = _celu(jnp.dot(h, a2g_ref[...], preferred_element_type=f32) + b2g_ref[...])
    vg = jnp.dot(h, a3g_ref[...], preferred_element_type=f32) + b3g_ref[...]

    oh8 = (jax.lax.broadcasted_iota(jnp.int32, (BLK, N_LIN), 1) == idx).astype(f32)
    h1_lin = jnp.dot(x, a1l_ref[...], preferred_element_type=f32) + b1l_ref[...]
    h1s = _celu(_select(h1_lin, oh8, 16))
    g2_lin = jnp.dot(h1s, w2bd_ref[...], preferred_element_type=f32) + b2bd_ref[...]
    h2s = _celu(_select(g2_lin, oh8, 32))
    vl_all = jnp.dot(h2s, a3l_ref[...], preferred_element_type=f32)
    vl = _select(vl_all, oh8, 64)
    vl = vl + jnp.dot(oh8, b3l8_ref[...], preferred_element_type=f32)

    dot2 = jnp.sum(vg * vl, axis=1, keepdims=True) ** 2
    r = v - vg - vl
    r2 = jnp.sum(r * r, axis=1, keepdims=True)
    oh64 = (jax.lax.broadcasted_iota(jnp.int32, (BLK, 64), 1) == key).astype(f32)

    ones_col = jnp.ones((BLK, 1), f32)
    acc_vg[...] += jax.lax.dot_general(oh64, vg, (((0,), (0,)), ((), ())),
                                       preferred_element_type=f32)
    acc_cnt[...] += jax.lax.dot_general(oh64, ones_col, (((0,), (0,)), ((), ())),
                                        preferred_element_type=f32)
    acc_orth[...] += jnp.sum(oh8 * dot2, axis=0, keepdims=True)
    acc_recon[...] += jnp.sum(oh8 * r2, axis=0, keepdims=True)

    @pl.when(i == nb - 1)
    def _fin():
        cntc = acc_cnt[...]
        rk = jax.lax.broadcasted_iota(jnp.int32, (64, N_LIN), 0)
        ck = jax.lax.broadcasted_iota(jnp.int32, (64, N_LIN), 1)
        sel_i = (jnp.bitwise_and(rk, N_LIN - 1) == ck).astype(jnp.float32)
        cnt_i = jax.lax.dot_general(cntc, sel_i, (((0,), (0,)), ((), ())),
                                    preferred_element_type=jnp.float32)
        loss_orth = jnp.sum(acc_orth[...] / cnt_i)
        loss_recon = jnp.sum(acc_recon[...] / (cnt_i * 64.0))

        mean = acc_vg[...] / cntc

        t_min = jnp.float32(T_VALS)
        t_max = jnp.float32(-1)
        cnt_t = []
        for j in range(T_VALS):
            cj = jnp.sum(cntc[j * N_LIN:(j + 1) * N_LIN, :])
            cnt_t.append(cj)
            t_min = jnp.where(cj > 0, jnp.minimum(t_min, float(j)), t_min)
            t_max = jnp.where(cj > 0, jnp.maximum(t_max, float(j)), t_max)
        max_t = t_max - t_min + 1.0

        loss_sim = jnp.float32(0.0)
        for j in range(T_VALS):
            V = mean[j * N_LIN:(j + 1) * N_LIN, :]
            diff = V[:, None, :] - V[None, :, :]
            d2 = jnp.sum(diff * diff, axis=-1)
            d = jnp.where(d2 > 0, jnp.sqrt(jnp.where(d2 > 0, d2, 1.0)), 0.0)
            lj = jnp.sum(d) / (N_LIN * (N_LIN - 1))
            in_range = jnp.logical_and(float(j) >= t_min, float(j) <= t_max)
            loss_sim = loss_sim + jnp.where(in_range, lj, 0.0)
        loss_sim = loss_sim / max_t

        recon_ref[...] = loss_recon.reshape(1, 1)
        orth_ref[...] = loss_orth.reshape(1, 1)
        sim_ref[...] = loss_sim.reshape(1, 1)


@jax.jit
def kernel(v, x, idx, t, W1g, b1g, W2g, b2g, W3g, b3g,
           W1l, b1l, W2l, b2l, W3l, b3l):
    n, d_in = x.shape
    f32 = jnp.float32
    nb = n // BLK

    key = (t.astype(jnp.int32) * N_LIN + idx.astype(jnp.int32)).reshape(n, 1)

    a1g = W1g.T
    a2g = W2g.T
    a3g = W3g.T
    a1l = W1l.reshape(N_LIN * 16, d_in).T
    b1c = b1l.reshape(1, N_LIN * 16)
    w2all = W2l.transpose(2, 0, 1).reshape(16, N_LIN * 32)
    b2c = b2l.reshape(1, N_LIN * 32)
    a3l = W3l.transpose(2, 0, 1).reshape(32, N_LIN * 64)
    b3l8 = b3l

    row_spec = pl.BlockSpec((BLK, 64), lambda i: (i, 0))
    key_spec = pl.BlockSpec((BLK, 1), lambda i: (i, 0))

    def full(shape):
        nd = len(shape)
        return pl.BlockSpec(shape, lambda i, _nd=nd: (0,) * _nd)

    out_shape = [jax.ShapeDtypeStruct((1, 1), f32)] * 3
    scalar_spec = pl.BlockSpec((1, 1), lambda i: (0, 0))

    recon, orth, sim = pl.pallas_call(
        _body,
        grid=(nb,),
        in_specs=[key_spec, row_spec, row_spec,
                  full((64, 16)), full((1, 16)), full((16, 32)), full((1, 32)),
                  full((32, 64)), full((1, 64)),
                  full((64, 128)), full((1, 128)), full((16, 256)),
                  full((1, 256)), full((32, 512)), full((8, 64))],
        out_specs=[scalar_spec] * 3,
        out_shape=out_shape,
        scratch_shapes=[pltpu.VMEM((64, 64), f32), pltpu.VMEM((64, 1), f32),
                        pltpu.VMEM((1, 8), f32), pltpu.VMEM((1, 8), f32)],
    )(key, x, v, a1g, b1g.reshape(1, 16), a2g, b2g.reshape(1, 32),
      a3g, b3g.reshape(1, 64), a1l, b1c, w2all, b2c, a3l, b3l8)

    return recon[0, 0], orth[0, 0], sim[0, 0]


# speedup vs baseline: 2.2757x; 2.2757x over previous
import functools

import jax
import jax.numpy as jnp
import numpy as np
from jax.experimental import pallas as pl
from jax.experimental.pallas import tpu as pltpu

N_LIN = 8
T_VALS = 8
BLK = 4096


def _celu(h):
    return jnp.where(h > 0, h, jnp.exp(h) - 1.0)


def _body(key_ref, x_ref, v_ref,
          a1g_ref, b1g_ref, a2g_ref, b2g_ref, a3g_ref, b3g_ref,
          a1l_ref, b1l_ref, w2bd_ref, b2bd_ref, a3l_ref, b3l8_ref,
          recon_ref, orth_ref, sim_ref,
          acc_vg, acc_cnt, acc_orth, acc_recon):
    i = pl.program_id(0)
    nb = pl.num_programs(0)
    f32 = jnp.float32

    @pl.when(i == 0)
    def _init():
        acc_vg[...] = jnp.zeros_like(acc_vg)
        acc_cnt[...] = jnp.zeros_like(acc_cnt)
        acc_orth[...] = jnp.zeros_like(acc_orth)
        acc_recon[...] = jnp.zeros_like(acc_recon)

    x = x_ref[...]
    v = v_ref[...]
    key = key_ref[...]
    idx = jnp.bitwise_and(key, N_LIN - 1)

    h = _celu(jnp.dot(x, a1g_ref[...], preferred_element_type=f32) + b1g_ref[...])
    h = _celu(jnp.dot(h, a2g_ref[...], preferred_element_type=f32) + b2g_ref[...])
    vg = jnp.dot(h, a3g_ref[...], preferred_element_type=f32) + b3g_ref[...]

    h1 = _celu(jnp.dot(x, a1l_ref[...], preferred_element_type=f32) + b1l_ref[...])
    h2 = _celu(jnp.dot(h1, w2bd_ref[...], preferred_element_type=f32) + b2bd_ref[...])
    col2 = jax.lax.broadcasted_iota(jnp.int32, h2.shape, 1)
    h2 = jnp.where((col2 // 32) == idx, h2, 0.0)
    vl = jnp.dot(h2, a3l_ref[...], preferred_element_type=f32)
    oh8 = (jax.lax.broadcasted_iota(jnp.int32, (BLK, N_LIN), 1) == idx).astype(f32)
    vl = vl + jnp.dot(oh8, b3l8_ref[...], preferred_element_type=f32)

    dot2 = jnp.sum(vg * vl, axis=1, keepdims=True) ** 2
    r = v - vg - vl
    r2 = jnp.sum(r * r, axis=1, keepdims=True)
    oh64 = (jax.lax.broadcasted_iota(jnp.int32, (BLK, 64), 1) == key).astype(f32)

    ones_col = jnp.ones((BLK, 1), f32)
    acc_vg[...] += jax.lax.dot_general(oh64, vg, (((0,), (0,)), ((), ())),
                                       preferred_element_type=f32)
    acc_cnt[...] += jax.lax.dot_general(oh64, ones_col, (((0,), (0,)), ((), ())),
                                        preferred_element_type=f32)
    acc_orth[...] += jnp.sum(oh8 * dot2, axis=0, keepdims=True)
    acc_recon[...] += jnp.sum(oh8 * r2, axis=0, keepdims=True)

    @pl.when(i == nb - 1)
    def _fin():
        cntc = acc_cnt[...]
        rk = jax.lax.broadcasted_iota(jnp.int32, (64, N_LIN), 0)
        ck = jax.lax.broadcasted_iota(jnp.int32, (64, N_LIN), 1)
        sel_i = (jnp.bitwise_and(rk, N_LIN - 1) == ck).astype(jnp.float32)
        cnt_i = jax.lax.dot_general(cntc, sel_i, (((0,), (0,)), ((), ())),
                                    preferred_element_type=jnp.float32)
        loss_orth = jnp.sum(acc_orth[...] / cnt_i)
        loss_recon = jnp.sum(acc_recon[...] / (cnt_i * 64.0))

        mean = acc_vg[...] / cntc

        t_min = jnp.float32(T_VALS)
        t_max = jnp.float32(-1)
        cnt_t = []
        for j in range(T_VALS):
            cj = jnp.sum(cntc[j * N_LIN:(j + 1) * N_LIN, :])
            cnt_t.append(cj)
            t_min = jnp.where(cj > 0, jnp.minimum(t_min, float(j)), t_min)
            t_max = jnp.where(cj > 0, jnp.maximum(t_max, float(j)), t_max)
        max_t = t_max - t_min + 1.0

        loss_sim = jnp.float32(0.0)
        for j in range(T_VALS):
            V = mean[j * N_LIN:(j + 1) * N_LIN, :]
            diff = V[:, None, :] - V[None, :, :]
            d2 = jnp.sum(diff * diff, axis=-1)
            d = jnp.where(d2 > 0, jnp.sqrt(jnp.where(d2 > 0, d2, 1.0)), 0.0)
            lj = jnp.sum(d) / (N_LIN * (N_LIN - 1))
            in_range = jnp.logical_and(float(j) >= t_min, float(j) <= t_max)
            loss_sim = loss_sim + jnp.where(in_range, lj, 0.0)
        loss_sim = loss_sim / max_t

        recon_ref[...] = loss_recon.reshape(1, 1)
        orth_ref[...] = loss_orth.reshape(1, 1)
        sim_ref[...] = loss_sim.reshape(1, 1)


@jax.jit
def kernel(v, x, idx, t, W1g, b1g, W2g, b2g, W3g, b3g,
           W1l, b1l, W2l, b2l, W3l, b3l):
    n, d_in = x.shape
    f32 = jnp.float32
    nb = n // BLK

    key = (t.astype(jnp.int32) * N_LIN + idx.astype(jnp.int32)).reshape(n, 1)

    a1g = W1g.T
    a2g = W2g.T
    a3g = W3g.T
    a1l = W1l.reshape(N_LIN * 16, d_in).T
    b1c = b1l.reshape(1, N_LIN * 16)
    w2bd = jnp.zeros((N_LIN, 16, N_LIN, 32), f32)
    w2bd = w2bd.at[jnp.arange(N_LIN), :, jnp.arange(N_LIN), :].set(
        W2l.transpose(0, 2, 1))
    w2bd = w2bd.reshape(N_LIN * 16, N_LIN * 32)
    b2c = b2l.reshape(1, N_LIN * 32)
    a3l = W3l.transpose(0, 2, 1).reshape(N_LIN * 32, 64)
    b3l8 = b3l

    row_spec = pl.BlockSpec((BLK, 64), lambda i: (i, 0))
    key_spec = pl.BlockSpec((BLK, 1), lambda i: (i, 0))

    def full(shape):
        nd = len(shape)
        return pl.BlockSpec(shape, lambda i, _nd=nd: (0,) * _nd)

    out_shape = [jax.ShapeDtypeStruct((1, 1), f32)] * 3
    scalar_spec = pl.BlockSpec((1, 1), lambda i: (0, 0))

    recon, orth, sim = pl.pallas_call(
        _body,
        grid=(nb,),
        in_specs=[key_spec, row_spec, row_spec,
                  full((64, 16)), full((1, 16)), full((16, 32)), full((1, 32)),
                  full((32, 64)), full((1, 64)),
                  full((64, 128)), full((1, 128)), full((128, 256)),
                  full((1, 256)), full((256, 64)), full((8, 64))],
        out_specs=[scalar_spec] * 3,
        out_shape=out_shape,
        scratch_shapes=[pltpu.VMEM((64, 64), f32), pltpu.VMEM((64, 1), f32),
                        pltpu.VMEM((1, 8), f32), pltpu.VMEM((1, 8), f32)],
    )(key, x, v, a1g, b1g.reshape(1, 16), a2g, b2g.reshape(1, 32),
      a3g, b3g.reshape(1, 64), a1l, b1c, w2bd, b2c, a3l, b3l8)

    return recon[0, 0], orth[0, 0], sim[0, 0]


# speedup vs baseline: 2.2937x; 1.0079x over previous
import functools

import jax
import jax.numpy as jnp
import numpy as np
from jax.experimental import pallas as pl
from jax.experimental.pallas import tpu as pltpu

N_LIN = 8
T_VALS = 8
BLK = 4096


def _celu(h):
    return jnp.where(h > 0, h, jnp.exp(h) - 1.0)


def _body(key_ref, x_ref, v_ref,
          a1g_ref, b1g_ref, a2g_ref, b2g_ref, a3g_ref, b3g_ref,
          a1l_ref, b1l_ref, w2bd_ref, b2bd_ref, a3l_ref, b3l8_ref,
          recon_ref, orth_ref, sim_ref,
          acc_vg, acc_cnt, acc_orth, acc_recon):
    i = pl.program_id(0)
    nb = pl.num_programs(0)
    f32 = jnp.float32

    @pl.when(i == 0)
    def _init():
        acc_vg[...] = jnp.zeros_like(acc_vg)
        acc_cnt[...] = jnp.zeros_like(acc_cnt)
        acc_orth[...] = jnp.zeros_like(acc_orth)
        acc_recon[...] = jnp.zeros_like(acc_recon)

    x = x_ref[...]
    v = v_ref[...]
    key = key_ref[...]
    idx = jnp.bitwise_and(key, N_LIN - 1)

    bf16 = jnp.bfloat16
    xb = x.astype(bf16)

    h = _celu(jnp.dot(xb, a1g_ref[...], preferred_element_type=f32) + b1g_ref[...])
    h = _celu(jnp.dot(h.astype(bf16), a2g_ref[...], preferred_element_type=f32)
              + b2g_ref[...])
    vg = jnp.dot(h.astype(bf16), a3g_ref[...], preferred_element_type=f32) + b3g_ref[...]

    h1 = _celu(jnp.dot(xb, a1l_ref[...], preferred_element_type=f32) + b1l_ref[...])
    h2 = _celu(jnp.dot(h1.astype(bf16), w2bd_ref[...], preferred_element_type=f32)
               + b2bd_ref[...])
    col2 = jax.lax.broadcasted_iota(jnp.int32, h2.shape, 1)
    h2 = jnp.where((col2 // 32) == idx, h2, 0.0)
    vl = jnp.dot(h2.astype(bf16), a3l_ref[...], preferred_element_type=f32)
    oh8 = (jax.lax.broadcasted_iota(jnp.int32, (BLK, N_LIN), 1) == idx).astype(f32)
    vl = vl + jnp.dot(oh8, b3l8_ref[...], preferred_element_type=f32)

    dot2 = jnp.sum(vg * vl, axis=1, keepdims=True) ** 2
    r = v - vg - vl
    r2 = jnp.sum(r * r, axis=1, keepdims=True)
    oh64 = (jax.lax.broadcasted_iota(jnp.int32, (BLK, 64), 1) == key).astype(f32)

    ones_col = jnp.ones((BLK, 1), f32)
    acc_vg[...] += jax.lax.dot_general(oh64, vg, (((0,), (0,)), ((), ())),
                                       preferred_element_type=f32)
    acc_cnt[...] += jax.lax.dot_general(oh64, ones_col, (((0,), (0,)), ((), ())),
                                        preferred_element_type=f32)
    acc_orth[...] += jnp.sum(oh8 * dot2, axis=0, keepdims=True)
    acc_recon[...] += jnp.sum(oh8 * r2, axis=0, keepdims=True)

    @pl.when(i == nb - 1)
    def _fin():
        cntc = acc_cnt[...]
        rk = jax.lax.broadcasted_iota(jnp.int32, (64, N_LIN), 0)
        ck = jax.lax.broadcasted_iota(jnp.int32, (64, N_LIN), 1)
        sel_i = (jnp.bitwise_and(rk, N_LIN - 1) == ck).astype(jnp.float32)
        cnt_i = jax.lax.dot_general(cntc, sel_i, (((0,), (0,)), ((), ())),
                                    preferred_element_type=jnp.float32)
        loss_orth = jnp.sum(acc_orth[...] / cnt_i)
        loss_recon = jnp.sum(acc_recon[...] / (cnt_i * 64.0))

        mean = acc_vg[...] / cntc

        t_min = jnp.float32(T_VALS)
        t_max = jnp.float32(-1)
        cnt_t = []
        for j in range(T_VALS):
            cj = jnp.sum(cntc[j * N_LIN:(j + 1) * N_LIN, :])
            cnt_t.append(cj)
            t_min = jnp.where(cj > 0, jnp.minimum(t_min, float(j)), t_min)
            t_max = jnp.where(cj > 0, jnp.maximum(t_max, float(j)), t_max)
        max_t = t_max - t_min + 1.0

        loss_sim = jnp.float32(0.0)
        for j in range(T_VALS):
            V = mean[j * N_LIN:(j + 1) * N_LIN, :]
            diff = V[:, None, :] - V[None, :, :]
            d2 = jnp.sum(diff * diff, axis=-1)
            d = jnp.where(d2 > 0, jnp.sqrt(jnp.where(d2 > 0, d2, 1.0)), 0.0)
            lj = jnp.sum(d) / (N_LIN * (N_LIN - 1))
            in_range = jnp.logical_and(float(j) >= t_min, float(j) <= t_max)
            loss_sim = loss_sim + jnp.where(in_range, lj, 0.0)
        loss_sim = loss_sim / max_t

        recon_ref[...] = loss_recon.reshape(1, 1)
        orth_ref[...] = loss_orth.reshape(1, 1)
        sim_ref[...] = loss_sim.reshape(1, 1)


@jax.jit
def kernel(v, x, idx, t, W1g, b1g, W2g, b2g, W3g, b3g,
           W1l, b1l, W2l, b2l, W3l, b3l):
    n, d_in = x.shape
    f32 = jnp.float32
    nb = n // BLK

    key = (t.astype(jnp.int32) * N_LIN + idx.astype(jnp.int32)).reshape(n, 1)

    bf16 = jnp.bfloat16
    a1g = W1g.T.astype(bf16)
    a2g = W2g.T.astype(bf16)
    a3g = W3g.T.astype(bf16)
    a1l = W1l.reshape(N_LIN * 16, d_in).T.astype(bf16)
    b1c = b1l.reshape(1, N_LIN * 16)
    w2bd = jnp.zeros((N_LIN, 16, N_LIN, 32), f32)
    w2bd = w2bd.at[jnp.arange(N_LIN), :, jnp.arange(N_LIN), :].set(
        W2l.transpose(0, 2, 1))
    w2bd = w2bd.reshape(N_LIN * 16, N_LIN * 32).astype(bf16)
    b2c = b2l.reshape(1, N_LIN * 32)
    a3l = W3l.transpose(0, 2, 1).reshape(N_LIN * 32, 64).astype(bf16)
    b3l8 = b3l

    row_spec = pl.BlockSpec((BLK, 64), lambda i: (i, 0))
    key_spec = pl.BlockSpec((BLK, 1), lambda i: (i, 0))

    def full(shape):
        nd = len(shape)
        return pl.BlockSpec(shape, lambda i, _nd=nd: (0,) * _nd)

    out_shape = [jax.ShapeDtypeStruct((1, 1), f32)] * 3
    scalar_spec = pl.BlockSpec((1, 1), lambda i: (0, 0))

    recon, orth, sim = pl.pallas_call(
        _body,
        grid=(nb,),
        in_specs=[key_spec, row_spec, row_spec,
                  full((64, 16)), full((1, 16)), full((16, 32)), full((1, 32)),
                  full((32, 64)), full((1, 64)),
                  full((64, 128)), full((1, 128)), full((128, 256)),
                  full((1, 256)), full((256, 64)), full((8, 64))],
        out_specs=[scalar_spec] * 3,
        out_shape=out_shape,
        scratch_shapes=[pltpu.VMEM((64, 64), f32), pltpu.VMEM((64, 1), f32),
                        pltpu.VMEM((1, 8), f32), pltpu.VMEM((1, 8), f32)],
    )(key, x, v, a1g, b1g.reshape(1, 16), a2g, b2g.reshape(1, 32),
      a3g, b3g.reshape(1, 64), a1l, b1c, w2bd, b2c, a3l, b3l8)

    return recon[0, 0], orth[0, 0], sim[0, 0]


# speedup vs baseline: 4.9342x; 2.1512x over previous
import jax
import jax.numpy as jnp
from jax.experimental import pallas as pl
from jax.experimental.pallas import tpu as pltpu

N_LIN = 8
T_VALS = 8
BLK = 4096


def _celu(h):
    return jnp.where(h > 0, h, jnp.exp(h) - 1.0)


def _body(key_ref, xt_ref, vt_ref,
          w1g_ref, b1g_ref, w2g_ref, b2g_ref, w3g_ref, b3g_ref,
          w1c_ref, b1c_ref, w2bd_ref, b2c_ref, a3c_ref, b3t_ref,
          recon_ref, orth_ref, sim_ref,
          acc_vg, acc_misc):
    i = pl.program_id(0)
    nb = pl.num_programs(0)
    f32 = jnp.float32
    bf16 = jnp.bfloat16

    @pl.when(i == 0)
    def _init():
        acc_vg[...] = jnp.zeros_like(acc_vg)
        acc_misc[...] = jnp.zeros_like(acc_misc)

    xt = xt_ref[...].astype(bf16)
    vt = vt_ref[...]
    key = key_ref[0]
    idx = jnp.bitwise_and(key, N_LIN - 1)

    hg = _celu(jnp.dot(w1g_ref[...], xt, preferred_element_type=f32) + b1g_ref[...])
    hg = _celu(jnp.dot(w2g_ref[...], hg.astype(bf16), preferred_element_type=f32)
               + b2g_ref[...])
    vg = jnp.dot(w3g_ref[...], hg.astype(bf16), preferred_element_type=f32) + b3g_ref[...]

    h1 = _celu(jnp.dot(w1c_ref[...], xt, preferred_element_type=f32) + b1c_ref[...])
    h2 = _celu(jnp.dot(w2bd_ref[...], h1.astype(bf16), preferred_element_type=f32)
               + b2c_ref[...])
    row2 = jax.lax.broadcasted_iota(jnp.int32, h2.shape, 0)
    h2 = jnp.where((row2 // 32) == idx, h2, 0.0)
    vl = jnp.dot(a3c_ref[...], h2.astype(bf16), preferred_element_type=f32)
    oh8 = (jax.lax.broadcasted_iota(jnp.int32, (N_LIN, BLK), 0) == idx).astype(f32)
    vl = vl + jnp.dot(b3t_ref[...], oh8, preferred_element_type=f32)

    dot2 = jnp.sum(vg * vl, axis=0, keepdims=True) ** 2
    r = vt - vg - vl
    r2 = jnp.sum(r * r, axis=0, keepdims=True)
    ones = jnp.ones((1, BLK), f32)
    misc = jnp.concatenate([dot2, r2, ones], axis=0)
    oh64 = (jax.lax.broadcasted_iota(jnp.int32, (64, BLK), 0) == key).astype(f32)

    acc_vg[...] += jax.lax.dot_general(oh64, vg, (((1,), (1,)), ((), ())),
                                       preferred_element_type=f32)
    acc_misc[...] += jax.lax.dot_general(oh64, misc, (((1,), (1,)), ((), ())),
                                         preferred_element_type=f32)

    @pl.when(i == nb - 1)
    def _fin():
        cntc = acc_misc[:, 2:3]
        rr = jax.lax.broadcasted_iota(jnp.int32, (N_LIN, 64), 0)
        cc = jax.lax.broadcasted_iota(jnp.int32, (N_LIN, 64), 1)
        s8 = (jnp.bitwise_and(cc, N_LIN - 1) == rr).astype(jnp.float32)
        per_lin = jnp.dot(s8, acc_misc[...],
                          preferred_element_type=jnp.float32)
        cnt_i = per_lin[:, 2:3]
        loss_orth = jnp.sum(per_lin[:, 0:1] / cnt_i)
        loss_recon = jnp.sum(per_lin[:, 1:2] / (cnt_i * 64.0))

        mean = acc_vg[...] / cntc

        t_min = jnp.float32(T_VALS)
        t_max = jnp.float32(-1)
        cs = []
        for j in range(T_VALS):
            cj = jnp.sum(cntc[j * N_LIN:(j + 1) * N_LIN, :])
            cs.append(cj)
            t_min = jnp.where(cj > 0, jnp.minimum(t_min, float(j)), t_min)
            t_max = jnp.where(cj > 0, jnp.maximum(t_max, float(j)), t_max)
        max_t = t_max - t_min + 1.0

        loss_sim = jnp.float32(0.0)
        for j in range(T_VALS):
            V = mean[j * N_LIN:(j + 1) * N_LIN, :]
            diff = V[:, None, :] - V[None, :, :]
            d2 = jnp.sum(diff * diff, axis=-1)
            d = jnp.where(d2 > 0, jnp.sqrt(jnp.where(d2 > 0, d2, 1.0)), 0.0)
            lj = jnp.sum(d) / (N_LIN * (N_LIN - 1))
            in_range = jnp.logical_and(float(j) >= t_min, float(j) <= t_max)
            loss_sim = loss_sim + jnp.where(in_range, lj, 0.0)
        loss_sim = loss_sim / max_t

        recon_ref[...] = loss_recon.reshape(1, 1)
        orth_ref[...] = loss_orth.reshape(1, 1)
        sim_ref[...] = loss_sim.reshape(1, 1)


@jax.jit
def kernel(v, x, idx, t, W1g, b1g, W2g, b2g, W3g, b3g,
           W1l, b1l, W2l, b2l, W3l, b3l):
    n, d_in = x.shape
    f32 = jnp.float32
    bf16 = jnp.bfloat16
    nb = n // BLK

    key3 = (t.astype(jnp.int32) * N_LIN + idx.astype(jnp.int32)).reshape(nb, 1, BLK)
    xt = x.T
    vt = v.T

    w1g = W1g.astype(bf16)
    w2g = W2g.astype(bf16)
    w3g = W3g.astype(bf16)
    w1c = W1l.reshape(N_LIN * 16, d_in).astype(bf16)
    b1c = b1l.reshape(N_LIN * 16, 1)
    w2bd = jnp.zeros((N_LIN, 32, N_LIN, 16), f32)
    w2bd = w2bd.at[jnp.arange(N_LIN), :, jnp.arange(N_LIN), :].set(W2l)
    w2bd = w2bd.reshape(N_LIN * 32, N_LIN * 16).astype(bf16)
    b2c = b2l.reshape(N_LIN * 32, 1)
    a3c = W3l.transpose(1, 0, 2).reshape(64, N_LIN * 32).astype(bf16)
    b3t = b3l.T

    row_spec = pl.BlockSpec((64, BLK), lambda i: (0, i))
    key_spec = pl.BlockSpec((1, 1, BLK), lambda i: (i, 0, 0))

    def full(shape):
        nd = len(shape)
        return pl.BlockSpec(shape, lambda i, _nd=nd: (0,) * _nd)

    out_shape = [jax.ShapeDtypeStruct((1, 1), f32)] * 3
    scalar_spec = pl.BlockSpec((1, 1), lambda i: (0, 0))

    recon, orth, sim = pl.pallas_call(
        _body,
        grid=(nb,),
        in_specs=[key_spec, row_spec, row_spec,
                  full((16, 64)), full((16, 1)), full((32, 16)), full((32, 1)),
                  full((64, 32)), full((64, 1)),
                  full((128, 64)), full((128, 1)), full((256, 128)),
                  full((256, 1)), full((64, 256)), full((64, 8))],
        out_specs=[scalar_spec] * 3,
        out_shape=out_shape,
        scratch_shapes=[pltpu.VMEM((64, 64), f32), pltpu.VMEM((64, 3), f32)],
    )(key3, xt, vt, w1g, b1g.reshape(16, 1), w2g, b2g.reshape(32, 1),
      w3g, b3g.reshape(64, 1), w1c, b1c, w2bd, b2c, a3c, b3t)

    return recon[0, 0], orth[0, 0], sim[0, 0]


# speedup vs baseline: 5.7542x; 1.1662x over previous
import jax
import jax.numpy as jnp
from jax.experimental import pallas as pl
from jax.experimental.pallas import tpu as pltpu

N_LIN = 8
T_VALS = 8
BLK = 4096


def _celu(h):
    return jnp.where(h > 0, h, jnp.exp(h) - 1.0)


def _body(key_ref, xt_ref, vt_ref,
          w1g_ref, b1g_ref, w2g_ref, b2g_ref, w3g_ref, b3g_ref,
          w1c_ref, b1c_ref, w2bd_ref, b2c_ref, a3c_ref, b3t_ref,
          recon_ref, orth_ref, sim_ref,
          acc_vg, acc_misc):
    i = pl.program_id(0)
    nb = pl.num_programs(0)
    f32 = jnp.float32
    bf16 = jnp.bfloat16

    @pl.when(i == 0)
    def _init():
        acc_vg[...] = jnp.zeros_like(acc_vg)
        acc_misc[...] = jnp.zeros_like(acc_misc)

    xt = xt_ref[...].astype(bf16)
    vt = vt_ref[...]
    key = key_ref[0]
    idx = jnp.bitwise_and(key, N_LIN - 1)

    hg = _celu(jnp.dot(w1g_ref[...], xt,
                       preferred_element_type=f32).astype(bf16) + b1g_ref[...])
    hg = _celu(jnp.dot(w2g_ref[...], hg,
                       preferred_element_type=f32).astype(bf16) + b2g_ref[...])
    vg = jnp.dot(w3g_ref[...], hg, preferred_element_type=f32) + b3g_ref[...]

    h1 = _celu(jnp.dot(w1c_ref[...], xt,
                       preferred_element_type=f32).astype(bf16) + b1c_ref[...])
    h2 = _celu(jnp.dot(w2bd_ref[...], h1,
                       preferred_element_type=f32).astype(bf16) + b2c_ref[...])
    row2 = jax.lax.broadcasted_iota(jnp.int32, h2.shape, 0)
    h2 = jnp.where((row2 // 32) == idx, h2, jnp.bfloat16(0.0))
    vl = jnp.dot(a3c_ref[...], h2, preferred_element_type=f32)
    oh8 = (jax.lax.broadcasted_iota(jnp.int32, (N_LIN, BLK), 0) == idx).astype(f32)
    vl = vl + jnp.dot(b3t_ref[...], oh8, preferred_element_type=f32)

    dot2 = jnp.sum(vg * vl, axis=0, keepdims=True) ** 2
    r = vt - vg - vl
    r2 = jnp.sum(r * r, axis=0, keepdims=True)
    ones = jnp.ones((1, BLK), f32)
    misc = jnp.concatenate([dot2, r2, ones], axis=0)
    oh64 = (jax.lax.broadcasted_iota(jnp.int32, (64, BLK), 0) == key).astype(f32)

    acc_vg[...] += jax.lax.dot_general(oh64, vg, (((1,), (1,)), ((), ())),
                                       preferred_element_type=f32)
    acc_misc[...] += jax.lax.dot_general(oh64, misc, (((1,), (1,)), ((), ())),
                                         preferred_element_type=f32)

    @pl.when(i == nb - 1)
    def _fin():
        cntc = acc_misc[:, 2:3]
        rr = jax.lax.broadcasted_iota(jnp.int32, (N_LIN, 64), 0)
        cc = jax.lax.broadcasted_iota(jnp.int32, (N_LIN, 64), 1)
        s8 = (jnp.bitwise_and(cc, N_LIN - 1) == rr).astype(jnp.float32)
        per_lin = jnp.dot(s8, acc_misc[...],
                          preferred_element_type=jnp.float32)
        cnt_i = per_lin[:, 2:3]
        loss_orth = jnp.sum(per_lin[:, 0:1] / cnt_i)
        loss_recon = jnp.sum(per_lin[:, 1:2] / (cnt_i * 64.0))

        mean = acc_vg[...] / cntc

        t_min = jnp.float32(T_VALS)
        t_max = jnp.float32(-1)
        cs = []
        for j in range(T_VALS):
            cj = jnp.sum(cntc[j * N_LIN:(j + 1) * N_LIN, :])
            cs.append(cj)
            t_min = jnp.where(cj > 0, jnp.minimum(t_min, float(j)), t_min)
            t_max = jnp.where(cj > 0, jnp.maximum(t_max, float(j)), t_max)
        max_t = t_max - t_min + 1.0

        loss_sim = jnp.float32(0.0)
        for j in range(T_VALS):
            V = mean[j * N_LIN:(j + 1) * N_LIN, :]
            diff = V[:, None, :] - V[None, :, :]
            d2 = jnp.sum(diff * diff, axis=-1)
            d = jnp.where(d2 > 0, jnp.sqrt(jnp.where(d2 > 0, d2, 1.0)), 0.0)
            lj = jnp.sum(d) / (N_LIN * (N_LIN - 1))
            in_range = jnp.logical_and(float(j) >= t_min, float(j) <= t_max)
            loss_sim = loss_sim + jnp.where(in_range, lj, 0.0)
        loss_sim = loss_sim / max_t

        recon_ref[...] = loss_recon.reshape(1, 1)
        orth_ref[...] = loss_orth.reshape(1, 1)
        sim_ref[...] = loss_sim.reshape(1, 1)


@jax.jit
def kernel(v, x, idx, t, W1g, b1g, W2g, b2g, W3g, b3g,
           W1l, b1l, W2l, b2l, W3l, b3l):
    n, d_in = x.shape
    f32 = jnp.float32
    bf16 = jnp.bfloat16
    nb = n // BLK

    key3 = (t.astype(jnp.int32) * N_LIN + idx.astype(jnp.int32)).reshape(nb, 1, BLK)
    xt = x.T
    vt = v.T

    w1g = W1g.astype(bf16)
    w2g = W2g.astype(bf16)
    w3g = W3g.astype(bf16)
    w1c = W1l.reshape(N_LIN * 16, d_in).astype(bf16)
    b1c = b1l.reshape(N_LIN * 16, 1).astype(bf16)
    w2bd = jnp.zeros((N_LIN, 32, N_LIN, 16), f32)
    w2bd = w2bd.at[jnp.arange(N_LIN), :, jnp.arange(N_LIN), :].set(W2l)
    w2bd = w2bd.reshape(N_LIN * 32, N_LIN * 16).astype(bf16)
    b2c = b2l.reshape(N_LIN * 32, 1).astype(bf16)
    a3c = W3l.transpose(1, 0, 2).reshape(64, N_LIN * 32).astype(bf16)
    b3t = b3l.T

    row_spec = pl.BlockSpec((64, BLK), lambda i: (0, i))
    key_spec = pl.BlockSpec((1, 1, BLK), lambda i: (i, 0, 0))

    def full(shape):
        nd = len(shape)
        return pl.BlockSpec(shape, lambda i, _nd=nd: (0,) * _nd)

    out_shape = [jax.ShapeDtypeStruct((1, 1), f32)] * 3
    scalar_spec = pl.BlockSpec((1, 1), lambda i: (0, 0))

    recon, orth, sim = pl.pallas_call(
        _body,
        grid=(nb,),
        in_specs=[key_spec, row_spec, row_spec,
                  full((16, 64)), full((16, 1)), full((32, 16)), full((32, 1)),
                  full((64, 32)), full((64, 1)),
                  full((128, 64)), full((128, 1)), full((256, 128)),
                  full((256, 1)), full((64, 256)), full((64, 8))],
        out_specs=[scalar_spec] * 3,
        out_shape=out_shape,
        scratch_shapes=[pltpu.VMEM((64, 64), f32), pltpu.VMEM((64, 3), f32)],
    )(key3, xt, vt, w1g, b1g.reshape(16, 1).astype(bf16), w2g,
      b2g.reshape(32, 1).astype(bf16),
      w3g, b3g.reshape(64, 1), w1c, b1c, w2bd, b2c, a3c, b3t)

    return recon[0, 0], orth[0, 0], sim[0, 0]


# speedup vs baseline: 6.1255x; 1.0645x over previous
import jax
import jax.numpy as jnp
from jax.experimental import pallas as pl
from jax.experimental.pallas import tpu as pltpu

N_LIN = 8
T_VALS = 8
BLK = 8192


def _celu(h):
    return jnp.where(h > 0, h, jnp.exp(h) - 1.0)


def _body(key_ref, xt_ref, vt_ref,
          w1g_ref, b1g_ref, w2g_ref, b2g_ref, w3g_ref, b3g_ref,
          w1c_ref, b1c_ref, w2bd_ref, b2c_ref, a3c_ref, b3t_ref,
          recon_ref, orth_ref, sim_ref,
          acc_vg, acc_misc):
    i = pl.program_id(0)
    nb = pl.num_programs(0)
    f32 = jnp.float32
    bf16 = jnp.bfloat16

    @pl.when(i == 0)
    def _init():
        acc_vg[...] = jnp.zeros_like(acc_vg)
        acc_misc[...] = jnp.zeros_like(acc_misc)

    xt = xt_ref[...].astype(bf16)
    vt = vt_ref[...]
    key = key_ref[0]
    idx = jnp.bitwise_and(key, N_LIN - 1)

    hg = _celu(jnp.dot(w1g_ref[...], xt,
                       preferred_element_type=f32).astype(bf16) + b1g_ref[...])
    hg = _celu(jnp.dot(w2g_ref[...], hg,
                       preferred_element_type=f32).astype(bf16) + b2g_ref[...])
    vg = jnp.dot(w3g_ref[...], hg, preferred_element_type=f32) + b3g_ref[...]

    h1 = _celu(jnp.dot(w1c_ref[...], xt,
                       preferred_element_type=f32).astype(bf16) + b1c_ref[...])
    h2 = _celu(jnp.dot(w2bd_ref[...], h1,
                       preferred_element_type=f32).astype(bf16) + b2c_ref[...])
    row2 = jax.lax.broadcasted_iota(jnp.int32, h2.shape, 0)
    h2 = jnp.where((row2 // 32) == idx, h2, jnp.bfloat16(0.0))
    vl = jnp.dot(a3c_ref[...], h2, preferred_element_type=f32)
    oh8 = (jax.lax.broadcasted_iota(jnp.int32, (N_LIN, BLK), 0) == idx).astype(f32)
    vl = vl + jnp.dot(b3t_ref[...], oh8, preferred_element_type=f32)

    dot2 = jnp.sum(vg * vl, axis=0, keepdims=True) ** 2
    r = vt - vg - vl
    r2 = jnp.sum(r * r, axis=0, keepdims=True)
    ones = jnp.ones((1, BLK), f32)
    misc = jnp.concatenate([dot2, r2, ones], axis=0)
    oh64 = (jax.lax.broadcasted_iota(jnp.int32, (64, BLK), 0) == key).astype(f32)

    acc_vg[...] += jax.lax.dot_general(oh64, vg, (((1,), (1,)), ((), ())),
                                       preferred_element_type=f32)
    acc_misc[...] += jax.lax.dot_general(oh64, misc, (((1,), (1,)), ((), ())),
                                         preferred_element_type=f32)

    @pl.when(i == nb - 1)
    def _fin():
        cntc = acc_misc[:, 2:3]
        rr = jax.lax.broadcasted_iota(jnp.int32, (N_LIN, 64), 0)
        cc = jax.lax.broadcasted_iota(jnp.int32, (N_LIN, 64), 1)
        s8 = (jnp.bitwise_and(cc, N_LIN - 1) == rr).astype(jnp.float32)
        per_lin = jnp.dot(s8, acc_misc[...],
                          preferred_element_type=jnp.float32)
        cnt_i = per_lin[:, 2:3]
        loss_orth = jnp.sum(per_lin[:, 0:1] / cnt_i)
        loss_recon = jnp.sum(per_lin[:, 1:2] / (cnt_i * 64.0))

        mean = acc_vg[...] / cntc

        t_min = jnp.float32(T_VALS)
        t_max = jnp.float32(-1)
        cs = []
        for j in range(T_VALS):
            cj = jnp.sum(cntc[j * N_LIN:(j + 1) * N_LIN, :])
            cs.append(cj)
            t_min = jnp.where(cj > 0, jnp.minimum(t_min, float(j)), t_min)
            t_max = jnp.where(cj > 0, jnp.maximum(t_max, float(j)), t_max)
        max_t = t_max - t_min + 1.0

        loss_sim = jnp.float32(0.0)
        for j in range(T_VALS):
            V = mean[j * N_LIN:(j + 1) * N_LIN, :]
            diff = V[:, None, :] - V[None, :, :]
            d2 = jnp.sum(diff * diff, axis=-1)
            d = jnp.where(d2 > 0, jnp.sqrt(jnp.where(d2 > 0, d2, 1.0)), 0.0)
            lj = jnp.sum(d) / (N_LIN * (N_LIN - 1))
            in_range = jnp.logical_and(float(j) >= t_min, float(j) <= t_max)
            loss_sim = loss_sim + jnp.where(in_range, lj, 0.0)
        loss_sim = loss_sim / max_t

        recon_ref[...] = loss_recon.reshape(1, 1)
        orth_ref[...] = loss_orth.reshape(1, 1)
        sim_ref[...] = loss_sim.reshape(1, 1)


@jax.jit
def kernel(v, x, idx, t, W1g, b1g, W2g, b2g, W3g, b3g,
           W1l, b1l, W2l, b2l, W3l, b3l):
    n, d_in = x.shape
    f32 = jnp.float32
    bf16 = jnp.bfloat16
    nb = n // BLK

    key3 = (t.astype(jnp.int32) * N_LIN + idx.astype(jnp.int32)).reshape(nb, 1, BLK)
    xt = x.T
    vt = v.T

    w1g = W1g.astype(bf16)
    w2g = W2g.astype(bf16)
    w3g = W3g.astype(bf16)
    w1c = W1l.reshape(N_LIN * 16, d_in).astype(bf16)
    b1c = b1l.reshape(N_LIN * 16, 1).astype(bf16)
    w2bd = jnp.zeros((N_LIN, 32, N_LIN, 16), f32)
    w2bd = w2bd.at[jnp.arange(N_LIN), :, jnp.arange(N_LIN), :].set(W2l)
    w2bd = w2bd.reshape(N_LIN * 32, N_LIN * 16).astype(bf16)
    b2c = b2l.reshape(N_LIN * 32, 1).astype(bf16)
    a3c = W3l.transpose(1, 0, 2).reshape(64, N_LIN * 32).astype(bf16)
    b3t = b3l.T

    row_spec = pl.BlockSpec((64, BLK), lambda i: (0, i))
    key_spec = pl.BlockSpec((1, 1, BLK), lambda i: (i, 0, 0))

    def full(shape):
        nd = len(shape)
        return pl.BlockSpec(shape, lambda i, _nd=nd: (0,) * _nd)

    out_shape = [jax.ShapeDtypeStruct((1, 1), f32)] * 3
    scalar_spec = pl.BlockSpec((1, 1), lambda i: (0, 0))

    recon, orth, sim = pl.pallas_call(
        _body,
        grid=(nb,),
        in_specs=[key_spec, row_spec, row_spec,
                  full((16, 64)), full((16, 1)), full((32, 16)), full((32, 1)),
                  full((64, 32)), full((64, 1)),
                  full((128, 64)), full((128, 1)), full((256, 128)),
                  full((256, 1)), full((64, 256)), full((64, 8))],
        out_specs=[scalar_spec] * 3,
        out_shape=out_shape,
        scratch_shapes=[pltpu.VMEM((64, 64), f32), pltpu.VMEM((64, 3), f32)],
    )(key3, xt, vt, w1g, b1g.reshape(16, 1).astype(bf16), w2g,
      b2g.reshape(32, 1).astype(bf16),
      w3g, b3g.reshape(64, 1), w1c, b1c, w2bd, b2c, a3c, b3t)

    return recon[0, 0], orth[0, 0], sim[0, 0]
